# baseline (speedup 1.0000x reference)
import numpy as np
import jax
import jax.numpy as jnp
from jax.experimental import pallas as pl

N = 62
F = 5
NH = 32
O1 = 64
NC = 3
NP = 64
NF = N * F
NFP = NP * F
NTRI = N * (N + 1) // 2
EWPAD = 2048
BN_EPS = 1e-5
_INV_SQRT1P = float(1.0 / np.sqrt(1.0 + BN_EPS))

_r = np.arange(NFP)
_c = np.arange(NFP)
_E_SEL = np.zeros((NFP, NP), np.float32)
_E_SEL[_r, _r // F] = 1.0
_MASK = ((_r[:, None] % F) == (_c[None, :] // O1)).astype(np.float32)
_JREP = ((_c[:, None] % O1) == np.arange(O1)[None, :]).astype(np.float32)
_TRI_STARTS = [i * (i + 1) // 2 for i in range(N)]

_INTERPRET = False


def _prep_kernel(ew_ref, gmat_ref, e_ref, mask_ref, jrep_ref, gt_ref, bt_ref,
                 bias1_ref, w_ref, c0_ref):
    col = jax.lax.broadcasted_iota(jnp.int32, (1, NP), 1)
    rows = []
    for i in range(N):
        seg = ew_ref[0:1, pl.ds(_TRI_STARTS[i], NP)]
        rows.append(jnp.where(col <= i, seg, 0.0))
    rows.append(jnp.zeros((NP - N, NP), jnp.float32))
    a0 = jnp.concatenate(rows, axis=0)
    eye = (jax.lax.broadcasted_iota(jnp.int32, (NP, NP), 0) ==
           jax.lax.broadcasted_iota(jnp.int32, (NP, NP), 1)).astype(jnp.float32)
    a0t = jax.lax.dot_general(a0, eye, (((0,), (0,)), ((), ())),
                              preferred_element_type=jnp.float32, precision=jax.lax.Precision.HIGHEST)
    a = a0 + a0t - a0 * eye
    a = jnp.maximum(a, 0.0)
    drow = jnp.sum(a, axis=1, keepdims=True)
    dcol = jnp.sum(a, axis=0, keepdims=True)
    l = a * jax.lax.rsqrt(drow + 1e-10) * jax.lax.rsqrt(dcol + 1e-10)
    la = jnp.abs(l)
    deg_r = jnp.sum(la, axis=1, keepdims=True) + 1.0
    deg_c = jnp.sum(la, axis=0, keepdims=True) + 1.0
    s = (l + eye) * jax.lax.rsqrt(deg_r) * jax.lax.rsqrt(deg_c)
    p = jnp.dot(s, s, preferred_element_type=jnp.float32, precision=jax.lax.Precision.HIGHEST)
    r = jnp.dot(p, gmat_ref[...], preferred_element_type=jnp.float32, precision=jax.lax.Precision.HIGHEST)
    t1 = jnp.dot(e_ref[...], r, preferred_element_type=jnp.float32, precision=jax.lax.Precision.HIGHEST)
    w0 = jnp.dot(t1 * mask_ref[...], jrep_ref[...],
                 preferred_element_type=jnp.float32, precision=jax.lax.Precision.HIGHEST)
    w_ref[...] = w0 * (gt_ref[...] * _INV_SQRT1P)
    c0_ref[...] = (jnp.dot(bt_ref[...], w0, preferred_element_type=jnp.float32, precision=jax.lax.Precision.HIGHEST)
                   + bias1_ref[...])


def _main_kernel(x_ref, w_ref, c0_ref, w2_ref, b2_ref, o_ref):
    w = w_ref[...]
    y = jnp.dot(x_ref[...], w[:NF, :], preferred_element_type=jnp.float32, precision=jax.lax.Precision.HIGHEST)
    y = jnp.maximum(y + c0_ref[...], 0.0)
    o_ref[...] = (jnp.dot(y, w2_ref[...], preferred_element_type=jnp.float32, precision=jax.lax.Precision.HIGHEST)
                  + b2_ref[...])


def kernel(X, edge_weight, bn_gamma, bn_beta, Wc, bc, W1, b1, W2, b2):
    B = X.shape[0]
    X_flat = X.reshape(B, NF)
    W1r = W1.reshape(O1, N, NH)
    G = jnp.einsum('onh,hf->nfo', W1r, Wc)
    Gmat = jnp.pad(G.reshape(N, F * O1), ((0, NP - N), (0, 0)))
    bias1 = (b1 + jnp.einsum('onh,h->o', W1r, bc)).reshape(1, O1)
    gt = jnp.pad(jnp.tile(bn_gamma, N), (0, NFP - NF)).reshape(NFP, 1)
    bt = jnp.pad(jnp.tile(bn_beta, N), (0, NFP - NF)).reshape(1, NFP)
    ew = jnp.pad(edge_weight, (0, EWPAD - NTRI)).reshape(1, EWPAD)

    wfold, c0 = pl.pallas_call(
        _prep_kernel,
        out_shape=[jax.ShapeDtypeStruct((NFP, O1), jnp.float32),
                   jax.ShapeDtypeStruct((1, O1), jnp.float32)],
        interpret=_INTERPRET,
    )(ew, Gmat, jnp.asarray(_E_SEL), jnp.asarray(_MASK), jnp.asarray(_JREP),
      gt, bt, bias1)

    BT = 256
    out = pl.pallas_call(
        _main_kernel,
        grid=(B // BT,),
        in_specs=[pl.BlockSpec((BT, NF), lambda i: (i, 0)),
                  pl.BlockSpec((NFP, O1), lambda i: (0, 0)),
                  pl.BlockSpec((1, O1), lambda i: (0, 0)),
                  pl.BlockSpec((O1, NC), lambda i: (0, 0)),
                  pl.BlockSpec((1, NC), lambda i: (0, 0))],
        out_specs=pl.BlockSpec((BT, NC), lambda i: (i, 0)),
        out_shape=jax.ShapeDtypeStruct((B, NC), jnp.float32),
        interpret=_INTERPRET,
    )(X_flat, wfold, c0, W2.T, b2.reshape(1, NC))
    return out


# speedup vs baseline: 1.0776x; 1.0776x over previous
import numpy as np
import jax
import jax.numpy as jnp
from jax.experimental import pallas as pl

N = 62
F = 5
NH = 32
O1 = 64
NC = 3
NP = 64
NF = N * F
NFP = NP * F
NTRI = N * (N + 1) // 2
EWPAD = 2048
BN_EPS = 1e-5
_INV_SQRT1P = float(1.0 / np.sqrt(1.0 + BN_EPS))

_r = np.arange(NFP)
_c = np.arange(NFP)
_E_SEL = np.zeros((NFP, NP), np.float32)
_E_SEL[_r, _r // F] = 1.0
_MASK = ((_r[:, None] % F) == (_c[None, :] // O1)).astype(np.float32)
_JREP = ((_c[:, None] % O1) == np.arange(O1)[None, :]).astype(np.float32)
_TRI_STARTS = [i * (i + 1) // 2 for i in range(N)]

_INTERPRET = False


def _prep_kernel(ew_ref, gmat_ref, e_ref, mask_ref, jrep_ref, gt_ref, bt_ref,
                 bias1_ref, w_ref, c0_ref):
    col = jax.lax.broadcasted_iota(jnp.int32, (1, NP), 1)
    rows = []
    for i in range(N):
        seg = ew_ref[0:1, pl.ds(_TRI_STARTS[i], NP)]
        rows.append(jnp.where(col <= i, seg, 0.0))
    rows.append(jnp.zeros((NP - N, NP), jnp.float32))
    a0 = jnp.concatenate(rows, axis=0)
    eye = (jax.lax.broadcasted_iota(jnp.int32, (NP, NP), 0) ==
           jax.lax.broadcasted_iota(jnp.int32, (NP, NP), 1)).astype(jnp.float32)
    a0t = jax.lax.dot_general(a0, eye, (((0,), (0,)), ((), ())),
                              preferred_element_type=jnp.float32, precision=jax.lax.Precision.HIGHEST)
    a = a0 + a0t - a0 * eye
    a = jnp.maximum(a, 0.0)
    drow = jnp.sum(a, axis=1, keepdims=True)
    dcol = jnp.sum(a, axis=0, keepdims=True)
    l = a * jax.lax.rsqrt(drow + 1e-10) * jax.lax.rsqrt(dcol + 1e-10)
    la = jnp.abs(l)
    deg_r = jnp.sum(la, axis=1, keepdims=True) + 1.0
    deg_c = jnp.sum(la, axis=0, keepdims=True) + 1.0
    s = (l + eye) * jax.lax.rsqrt(deg_r) * jax.lax.rsqrt(deg_c)
    p = jnp.dot(s, s, preferred_element_type=jnp.float32, precision=jax.lax.Precision.HIGHEST)
    r = jnp.dot(p, gmat_ref[...], preferred_element_type=jnp.float32, precision=jax.lax.Precision.HIGHEST)
    t1 = jnp.dot(e_ref[...], r, preferred_element_type=jnp.float32, precision=jax.lax.Precision.HIGHEST)
    w0 = jnp.dot(t1 * mask_ref[...], jrep_ref[...],
                 preferred_element_type=jnp.float32, precision=jax.lax.Precision.HIGHEST)
    w_ref[...] = w0 * (gt_ref[...] * _INV_SQRT1P)
    c0_ref[...] = (jnp.dot(bt_ref[...], w0, preferred_element_type=jnp.float32, precision=jax.lax.Precision.HIGHEST)
                   + bias1_ref[...])


def _main_kernel(x_ref, w_ref, c0_ref, w2_ref, b2_ref, o_ref):
    w = w_ref[...]
    y = jnp.dot(x_ref[...], w[:NF, :], preferred_element_type=jnp.float32)
    y = jnp.maximum(y + c0_ref[...], 0.0)
    o_ref[...] = (jnp.dot(y, w2_ref[...], preferred_element_type=jnp.float32)
                  + b2_ref[...])


def kernel(X, edge_weight, bn_gamma, bn_beta, Wc, bc, W1, b1, W2, b2):
    B = X.shape[0]
    X_flat = X.reshape(B, NF)
    W1r = W1.reshape(O1, N, NH)
    G = jnp.einsum('onh,hf->nfo', W1r, Wc)
    Gmat = jnp.pad(G.reshape(N, F * O1), ((0, NP - N), (0, 0)))
    bias1 = (b1 + jnp.einsum('onh,h->o', W1r, bc)).reshape(1, O1)
    gt = jnp.pad(jnp.tile(bn_gamma, N), (0, NFP - NF)).reshape(NFP, 1)
    bt = jnp.pad(jnp.tile(bn_beta, N), (0, NFP - NF)).reshape(1, NFP)
    ew = jnp.pad(edge_weight, (0, EWPAD - NTRI)).reshape(1, EWPAD)

    wfold, c0 = pl.pallas_call(
        _prep_kernel,
        out_shape=[jax.ShapeDtypeStruct((NFP, O1), jnp.float32),
                   jax.ShapeDtypeStruct((1, O1), jnp.float32)],
        interpret=_INTERPRET,
    )(ew, Gmat, jnp.asarray(_E_SEL), jnp.asarray(_MASK), jnp.asarray(_JREP),
      gt, bt, bias1)

    BT = 256
    out = pl.pallas_call(
        _main_kernel,
        grid=(B // BT,),
        in_specs=[pl.BlockSpec((BT, NF), lambda i: (i, 0)),
                  pl.BlockSpec((NFP, O1), lambda i: (0, 0)),
                  pl.BlockSpec((1, O1), lambda i: (0, 0)),
                  pl.BlockSpec((O1, NC), lambda i: (0, 0)),
                  pl.BlockSpec((1, NC), lambda i: (0, 0))],
        out_specs=pl.BlockSpec((BT, NC), lambda i: (i, 0)),
        out_shape=jax.ShapeDtypeStruct((B, NC), jnp.float32),
        interpret=_INTERPRET,
    )(X_flat, wfold, c0, W2.T, b2.reshape(1, NC))
    return out


# speedup vs baseline: 1.1704x; 1.0861x over previous
import numpy as np
import jax
import jax.numpy as jnp
from jax.experimental import pallas as pl
from jax.experimental.pallas import tpu as pltpu

N = 62
F = 5
NH = 32
O1 = 64
NC = 3
NP = 64
NF = N * F
NFP = NP * F
NTRI = N * (N + 1) // 2
EWPAD = 2048
BN_EPS = 1e-5
_INV_SQRT1P = float(1.0 / np.sqrt(1.0 + BN_EPS))
_HI = jax.lax.Precision.HIGHEST

_r = np.arange(NFP)
_c = np.arange(NFP)
_E_SEL = np.zeros((NFP, NP), np.float32)
_E_SEL[_r, _r // F] = 1.0
_MASK = ((_r[:, None] % F) == (_c[None, :] // O1)).astype(np.float32)
_JREP = ((_c[:, None] % O1) == np.arange(O1)[None, :]).astype(np.float32)
_TRI_STARTS = [i * (i + 1) // 2 for i in range(N)]

_INTERPRET = False


def _fold(ew_ref, gmat_ref, e_ref, mask_ref, jrep_ref, gt_ref, bt_ref,
          bias1_ref, w_acc, c0_acc):
    col = jax.lax.broadcasted_iota(jnp.int32, (1, NP), 1)
    rows = []
    for i in range(N):
        seg = ew_ref[0:1, pl.ds(_TRI_STARTS[i], NP)]
        rows.append(jnp.where(col <= i, seg, 0.0))
    rows.append(jnp.zeros((NP - N, NP), jnp.float32))
    a0 = jnp.concatenate(rows, axis=0)
    eye = (jax.lax.broadcasted_iota(jnp.int32, (NP, NP), 0) ==
           jax.lax.broadcasted_iota(jnp.int32, (NP, NP), 1)).astype(jnp.float32)
    a0t = jax.lax.dot_general(a0, eye, (((0,), (0,)), ((), ())),
                              preferred_element_type=jnp.float32, precision=_HI)
    a = a0 + a0t - a0 * eye
    a = jnp.maximum(a, 0.0)
    drow = jnp.sum(a, axis=1, keepdims=True)
    dcol = jnp.sum(a, axis=0, keepdims=True)
    l = a * jax.lax.rsqrt(drow + 1e-10) * jax.lax.rsqrt(dcol + 1e-10)
    la = jnp.abs(l)
    deg_r = jnp.sum(la, axis=1, keepdims=True) + 1.0
    deg_c = jnp.sum(la, axis=0, keepdims=True) + 1.0
    s = (l + eye) * jax.lax.rsqrt(deg_r) * jax.lax.rsqrt(deg_c)
    p = jnp.dot(s, s, preferred_element_type=jnp.float32, precision=_HI)
    r = jnp.dot(p, gmat_ref[...], preferred_element_type=jnp.float32, precision=_HI)
    t1 = jnp.dot(e_ref[...], r, preferred_element_type=jnp.float32, precision=_HI)
    w0 = jnp.dot(t1 * mask_ref[...], jrep_ref[...],
                 preferred_element_type=jnp.float32, precision=_HI)
    w_acc[...] = w0 * (gt_ref[...] * _INV_SQRT1P)
    c0_acc[...] = (jnp.dot(bt_ref[...], w0, preferred_element_type=jnp.float32,
                           precision=_HI)
                   + bias1_ref[...])


def _fused_kernel(ew_ref, gmat_ref, e_ref, mask_ref, jrep_ref, gt_ref, bt_ref,
                  bias1_ref, x_ref, w2_ref, b2_ref, o_ref, w_acc, c0_acc):
    @pl.when(pl.program_id(0) == 0)
    def _():
        _fold(ew_ref, gmat_ref, e_ref, mask_ref, jrep_ref, gt_ref, bt_ref,
              bias1_ref, w_acc, c0_acc)

    w = w_acc[...]
    y = jnp.dot(x_ref[...], w[:NF, :], preferred_element_type=jnp.float32)
    y = jnp.maximum(y + c0_acc[...], 0.0)
    o_ref[...] = (jnp.dot(y, w2_ref[...], preferred_element_type=jnp.float32)
                  + b2_ref[...])


def kernel(X, edge_weight, bn_gamma, bn_beta, Wc, bc, W1, b1, W2, b2):
    B = X.shape[0]
    X_flat = X.reshape(B, NF)
    W1r = W1.reshape(O1, N, NH)
    G = jnp.einsum('onh,hf->nfo', W1r, Wc)
    Gmat = jnp.pad(G.reshape(N, F * O1), ((0, NP - N), (0, 0)))
    bias1 = (b1 + jnp.einsum('onh,h->o', W1r, bc)).reshape(1, O1)
    gt = jnp.pad(jnp.tile(bn_gamma, N), (0, NFP - NF)).reshape(NFP, 1)
    bt = jnp.pad(jnp.tile(bn_beta, N), (0, NFP - NF)).reshape(1, NFP)
    ew = jnp.pad(edge_weight, (0, EWPAD - NTRI)).reshape(1, EWPAD)

    BT = 256
    cblk = lambda i: (0, 0)
    out = pl.pallas_call(
        _fused_kernel,
        grid=(B // BT,),
        in_specs=[pl.BlockSpec((1, EWPAD), cblk),
                  pl.BlockSpec((NP, F * O1), cblk),
                  pl.BlockSpec((NFP, NP), cblk),
                  pl.BlockSpec((NFP, NFP), cblk),
                  pl.BlockSpec((NFP, O1), cblk),
                  pl.BlockSpec((NFP, 1), cblk),
                  pl.BlockSpec((1, NFP), cblk),
                  pl.BlockSpec((1, O1), cblk),
                  pl.BlockSpec((BT, NF), lambda i: (i, 0)),
                  pl.BlockSpec((O1, NC), cblk),
                  pl.BlockSpec((1, NC), cblk)],
        out_specs=pl.BlockSpec((BT, NC), lambda i: (i, 0)),
        out_shape=jax.ShapeDtypeStruct((B, NC), jnp.float32),
        scratch_shapes=[pltpu.VMEM((NFP, O1), jnp.float32),
                        pltpu.VMEM((1, O1), jnp.float32)],
        interpret=_INTERPRET,
    )(ew, Gmat, jnp.asarray(_E_SEL), jnp.asarray(_MASK), jnp.asarray(_JREP),
      gt, bt, bias1, X_flat, W2.T, b2.reshape(1, NC))
    return out
